# baseline (speedup 1.0000x reference)
import functools

import jax
import jax.numpy as jnp
from jax import lax
from jax.experimental import pallas as pl
from jax.experimental.pallas import tpu as pltpu
from jax.experimental.pallas import tpu_sc as plsc

KNN_K = 27
_BIG = 1e30


def _embed_body(x_ref, we_ref, w1_ref, b1_ref, vw_ref, vb_ref, o_ref):
    h0 = jnp.dot(x_ref[...], we_ref[...], preferred_element_type=jnp.float32)
    t = jnp.tanh(jnp.dot(h0, w1_ref[...], preferred_element_type=jnp.float32)
                 + b1_ref[...])
    s0 = jnp.dot(t, vw_ref[...], preferred_element_type=jnp.float32) + vb_ref[...]
    n = h0.shape[0]
    o_ref[...] = jnp.concatenate(
        [h0, s0, jnp.zeros((n, 111), jnp.float32)], axis=1)


def _embed(x, W_embed, W1, b1, Vw, Vb):
    n = x.shape[0]
    return pl.pallas_call(
        _embed_body,
        out_shape=jax.ShapeDtypeStruct((n, 128), jnp.float32),
    )(x, W_embed, W1, b1.reshape(1, -1), Vw, Vb.reshape(1, 1))


def _sc_gather_rows(table, idx_flat):
    nidx = idx_flat.shape[0]
    n_chunk = -(-nidx // 4096)
    bpad = n_chunk * 4096
    idx2 = jnp.concatenate(
        [idx_flat, jnp.zeros((bpad - nidx,), jnp.int32)]).reshape(32 * n_chunk, 128)

    mesh = plsc.VectorSubcoreMesh(core_axis_name="c", subcore_axis_name="s")

    @functools.partial(
        pl.kernel, mesh=mesh,
        out_type=jax.ShapeDtypeStruct((bpad, 128), jnp.float32),
        scratch_types=[
            pltpu.VMEM((128,), jnp.int32),
            pltpu.VMEM((128, 128), jnp.float32),
            pltpu.SemaphoreType.DMA,
        ],
    )
    def gk(table_hbm, idx_hbm, out_hbm, idx_v, rows_v, sem):
        wid = lax.axis_index("s") * 2 + lax.axis_index("c")

        def body(c, carry):
            row = wid * n_chunk + c
            pltpu.sync_copy(idx_hbm.at[row], idx_v)
            pltpu.async_copy(table_hbm.at[idx_v], rows_v, sem).wait()
            pltpu.sync_copy(rows_v, out_hbm.at[pl.ds(row * 128, 128)])
            return carry

        lax.fori_loop(0, n_chunk, body, 0)

    return gk(table, idx2)


def _pool_body(*refs):
    g = refs[:-8]
    wq_ref, bq_ref, wk_ref, bk_ref, wv_ref, bv_ref, hq_ref, kv_ref = refs[-8:]
    gv = [r[...] for r in g]
    s = [gj[:, 16:17] for gj in gv]
    m = s[0]
    for sj in s[1:]:
        m = jnp.maximum(m, sj)
    e = [jnp.exp(sj - m) for sj in s]
    z = e[0]
    for ej in e[1:]:
        z = z + ej
    acc = e[0] * gv[0][:, 0:16]
    for ej, gj in zip(e[1:], gv[1:]):
        acc = acc + ej * gj[:, 0:16]
    h1 = acc / z
    q = jnp.dot(h1, wq_ref[...], preferred_element_type=jnp.float32) + bq_ref[...]
    k = jnp.dot(h1, wk_ref[...], preferred_element_type=jnp.float32) + bk_ref[...]
    v = jnp.dot(h1, wv_ref[...], preferred_element_type=jnp.float32) + bv_ref[...]
    pb = h1.shape[0]
    hq_ref[...] = jnp.concatenate([h1, q], axis=1)
    kv_ref[...] = jnp.concatenate(
        [k, v, jnp.zeros((pb, 96), jnp.float32)], axis=1)


def _pool_proj(g, kp, n, Wq, bq, Wk, bk, Wv, bv, pb):
    nb = n // pb
    wspec = pl.BlockSpec((16, 16), lambda i: (0, 0))
    bspec = pl.BlockSpec((1, 16), lambda i: (0, 0))
    gspecs = [pl.BlockSpec((pb, 128), functools.partial(
        lambda i, j: (j * nb + i, 0), j=j)) for j in range(kp)]
    return pl.pallas_call(
        _pool_body,
        grid=(nb,),
        in_specs=gspecs + [wspec, bspec, wspec, bspec, wspec, bspec],
        out_specs=[
            pl.BlockSpec((pb, 32), lambda i: (i, 0)),
            pl.BlockSpec((pb, 128), lambda i: (i, 0)),
        ],
        out_shape=[
            jax.ShapeDtypeStruct((n, 32), jnp.float32),
            jax.ShapeDtypeStruct((n, 128), jnp.float32),
        ],
    )(*([g] * kp), Wq, bq.reshape(1, -1), Wk, bk.reshape(1, -1),
      Wv, bv.reshape(1, -1))


def _topk_body(q_ref, xvt_ref, o_ref, *, k, npad, rounds):
    qb = q_ref[...]
    xvt = xvt_ref[...]
    qn = qb.shape[0]
    sq = jnp.sum(xvt * xvt, axis=0, keepdims=True)
    qsq = jnp.sum(qb * qb, axis=1, keepdims=True)
    d2 = qsq - 2.0 * jnp.dot(qb, xvt, preferred_element_type=jnp.float32) + sq
    nc = npad // 128
    d2r = d2.reshape(qn, nc, 128)
    ig = (lax.broadcasted_iota(jnp.int32, (1, nc, 128), 1) * 128
          + lax.broadcasted_iota(jnp.int32, (1, nc, 128), 2)).astype(jnp.float32)
    vals, idxs = [], []
    for _ in range(rounds):
        mc = jnp.min(d2r, axis=1, keepdims=True)
        cand = jnp.where(d2r == mc, ig, _BIG)
        ic = jnp.min(cand, axis=1, keepdims=True)
        vals.append(mc.reshape(qn, 128))
        idxs.append(ic.reshape(qn, 128))
        d2r = jnp.where(ig == ic, _BIG, d2r)
    v = jnp.concatenate(vals, axis=1)
    iv = jnp.concatenate(idxs, axis=1)
    cols = []
    for _ in range(k):
        m = jnp.min(v, axis=1, keepdims=True)
        cand = jnp.where(v == m, iv, _BIG)
        idxf = jnp.min(cand, axis=1, keepdims=True)
        cols.append(idxf)
        v = jnp.where(iv == idxf, _BIG, v)
    cols.append(jnp.zeros((qn, 32 - k), jnp.float32))
    o_ref[...] = jnp.concatenate(cols, axis=1).astype(jnp.int32)


def _knn_topk(xvp, xvt, qb, npad):
    grid = npad // qb
    return pl.pallas_call(
        functools.partial(_topk_body, k=KNN_K, npad=npad, rounds=5),
        grid=(grid,),
        in_specs=[
            pl.BlockSpec((qb, 3), lambda i: (i, 0)),
            pl.BlockSpec((3, npad), lambda i: (0, 0)),
        ],
        out_specs=pl.BlockSpec((qb, 32), lambda i: (i, 0)),
        out_shape=jax.ShapeDtypeStruct((npad, 32), jnp.int32),
    )(xvp, xvt)


def _mha_body(*refs):
    kv = [r[...] for r in refs[:KNN_K]]
    hq_ref, wo_ref, bo_ref, wout_ref, o_ref = refs[KNN_K:]
    hq = hq_ref[...]
    q = hq[:, 16:32]
    s = [q * kvj[:, 0:16] for kvj in kv]
    m = s[0]
    for sj in s[1:]:
        m = jnp.maximum(m, sj)
    e = [jnp.exp(sj - m) for sj in s]
    z = e[0]
    for ej in e[1:]:
        z = z + ej
    acc = e[0] * kv[0][:, 16:32]
    for ej, kvj in zip(e[1:], kv[1:]):
        acc = acc + ej * kvj[:, 16:32]
    o = acc / z
    res = hq[:, 0:16] + jnp.dot(o, wo_ref[...],
                                preferred_element_type=jnp.float32) + bo_ref[...]
    o_ref[...] = jnp.dot(res, wout_ref[...], preferred_element_type=jnp.float32)


def _mha_out(kvg, hq, Wo, bo, Wout, pb):
    n = hq.shape[0]
    nb = n // pb
    kvspecs = [pl.BlockSpec((pb, 128), functools.partial(
        lambda i, j: (j * nb + i, 0), j=j)) for j in range(KNN_K)]
    return pl.pallas_call(
        _mha_body,
        grid=(nb,),
        in_specs=kvspecs + [
            pl.BlockSpec((pb, 32), lambda i: (i, 0)),
            pl.BlockSpec((16, 16), lambda i: (0, 0)),
            pl.BlockSpec((1, 16), lambda i: (0, 0)),
            pl.BlockSpec((16, 1), lambda i: (0, 0)),
        ],
        out_specs=pl.BlockSpec((pb, 1), lambda i: (i, 0)),
        out_shape=jax.ShapeDtypeStruct((n, 1), jnp.float32),
    )(*([kvg] * KNN_K), hq, Wo, bo.reshape(1, -1), Wout)


def kernel(x, x_v, unpooling_idx, W_embed, W1, b1, Vw, Vb,
           Wq, bq, Wk, bk, Wv, bv, Wo, bo, Wout):
    n_in = x.shape[1]
    n_out = x_v.shape[1]
    kp = unpooling_idx.shape[2]

    x2 = x.reshape(n_in, 3)
    xv2 = x_v.reshape(n_out, 3)

    h0s = _embed(x2, W_embed, W1, b1, Vw, Vb)

    uidx = jnp.transpose(unpooling_idx.reshape(n_out, kp)).reshape(
        n_out * kp).astype(jnp.int32)
    g = _sc_gather_rows(h0s, uidx)

    pb = 400 if n_out % 400 == 0 else n_out
    hq, kv = _pool_proj(g, kp, n_out, Wq, bq, Wk, bk, Wv, bv, pb)

    qb = 128
    npad = -(-n_out // 512) * 512
    pad = jnp.full((npad - n_out, 3), 1e4, jnp.float32)
    xvp = jnp.concatenate([xv2, pad], axis=0)
    xvt = xvp.T
    knn = _knn_topk(xvp, xvt, qb, npad)
    kidx = jnp.transpose(knn[:n_out, :KNN_K]).reshape(
        KNN_K * n_out)

    kvg = _sc_gather_rows(kv, kidx)

    out = _mha_out(kvg, hq, Wo, bo, Wout, pb)
    return out.reshape(1, n_out, 1)


# speedup vs baseline: 1.2125x; 1.2125x over previous
import functools

import jax
import jax.numpy as jnp
from jax import lax
from jax.experimental import pallas as pl
from jax.experimental.pallas import tpu as pltpu
from jax.experimental.pallas import tpu_sc as plsc

KNN_K = 27
_BIG = 1e30


def _embed_body(x_ref, we_ref, w1_ref, b1_ref, vw_ref, vb_ref, o_ref):
    h0 = jnp.dot(x_ref[...], we_ref[...], preferred_element_type=jnp.float32)
    t = jnp.tanh(jnp.dot(h0, w1_ref[...], preferred_element_type=jnp.float32)
                 + b1_ref[...])
    s0 = jnp.dot(t, vw_ref[...], preferred_element_type=jnp.float32) + vb_ref[...]
    n = h0.shape[0]
    o_ref[...] = jnp.concatenate(
        [h0, s0, jnp.zeros((n, 111), jnp.float32)], axis=1)


def _embed(x, W_embed, W1, b1, Vw, Vb):
    n = x.shape[0]
    return pl.pallas_call(
        _embed_body,
        out_shape=jax.ShapeDtypeStruct((n, 128), jnp.float32),
    )(x, W_embed, W1, b1.reshape(1, -1), Vw, Vb.reshape(1, 1))


def _sc_gather_rows(table, idx_flat):
    nidx = idx_flat.shape[0]
    n_chunk = -(-nidx // 4096)
    bpad = n_chunk * 4096
    idx2 = jnp.concatenate(
        [idx_flat, jnp.zeros((bpad - nidx,), jnp.int32)]).reshape(32 * n_chunk, 128)

    mesh = plsc.VectorSubcoreMesh(core_axis_name="c", subcore_axis_name="s")

    @functools.partial(
        pl.kernel, mesh=mesh,
        out_type=jax.ShapeDtypeStruct((bpad, 128), jnp.float32),
        scratch_types=[
            pltpu.VMEM((2, 128), jnp.int32),
            pltpu.VMEM((2, 128, 128), jnp.float32),
            pltpu.SemaphoreType.DMA,
            pltpu.SemaphoreType.DMA,
        ],
    )
    def gk(table_hbm, idx_hbm, out_hbm, idx_v, rows_v, s_g, s_o):
        wid = lax.axis_index("s") * 2 + lax.axis_index("c")
        base = wid * n_chunk

        pltpu.sync_copy(idx_hbm.at[base], idx_v.at[0])
        pltpu.async_copy(table_hbm.at[idx_v.at[0]], rows_v.at[0], s_g)

        def body(c, carry):
            cur = lax.rem(c, 2)
            nxt = 1 - cur

            @pl.when(c + 1 < n_chunk)
            def _():
                @pl.when(c >= 1)
                def _():
                    pltpu.make_async_copy(
                        rows_v.at[nxt],
                        out_hbm.at[pl.ds((base + c - 1) * 128, 128)],
                        s_o).wait()
                pltpu.sync_copy(idx_hbm.at[base + c + 1], idx_v.at[nxt])
                pltpu.async_copy(
                    table_hbm.at[idx_v.at[nxt]], rows_v.at[nxt], s_g)

            pltpu.make_async_copy(
                table_hbm.at[idx_v.at[cur]], rows_v.at[cur], s_g).wait()
            pltpu.async_copy(
                rows_v.at[cur], out_hbm.at[pl.ds((base + c) * 128, 128)], s_o)
            return carry

        lax.fori_loop(0, n_chunk, body, 0)
        last = lax.rem(n_chunk - 1, 2)
        pltpu.make_async_copy(
            rows_v.at[1 - last],
            out_hbm.at[pl.ds((base + n_chunk - 2) * 128, 128)], s_o).wait()
        pltpu.make_async_copy(
            rows_v.at[last],
            out_hbm.at[pl.ds((base + n_chunk - 1) * 128, 128)], s_o).wait()

    return gk(table, idx2)


def _pool_body(*refs):
    g = refs[:-8]
    wq_ref, bq_ref, wk_ref, bk_ref, wv_ref, bv_ref, hq_ref, kv_ref = refs[-8:]
    gv = [r[...] for r in g]
    s = [gj[:, 16:17] for gj in gv]
    m = s[0]
    for sj in s[1:]:
        m = jnp.maximum(m, sj)
    e = [jnp.exp(sj - m) for sj in s]
    z = e[0]
    for ej in e[1:]:
        z = z + ej
    acc = e[0] * gv[0][:, 0:16]
    for ej, gj in zip(e[1:], gv[1:]):
        acc = acc + ej * gj[:, 0:16]
    h1 = acc / z
    q = jnp.dot(h1, wq_ref[...], preferred_element_type=jnp.float32) + bq_ref[...]
    k = jnp.dot(h1, wk_ref[...], preferred_element_type=jnp.float32) + bk_ref[...]
    v = jnp.dot(h1, wv_ref[...], preferred_element_type=jnp.float32) + bv_ref[...]
    pb = h1.shape[0]
    hq_ref[...] = jnp.concatenate([h1, q], axis=1)
    kv_ref[...] = jnp.concatenate(
        [k, v, jnp.zeros((pb, 96), jnp.float32)], axis=1)


def _pool_proj(g, kp, n, Wq, bq, Wk, bk, Wv, bv, pb):
    nb = n // pb
    wspec = pl.BlockSpec((16, 16), lambda i: (0, 0))
    bspec = pl.BlockSpec((1, 16), lambda i: (0, 0))
    gspecs = [pl.BlockSpec((pb, 128), functools.partial(
        lambda i, j: (j * nb + i, 0), j=j)) for j in range(kp)]
    return pl.pallas_call(
        _pool_body,
        grid=(nb,),
        in_specs=gspecs + [wspec, bspec, wspec, bspec, wspec, bspec],
        out_specs=[
            pl.BlockSpec((pb, 32), lambda i: (i, 0)),
            pl.BlockSpec((pb, 128), lambda i: (i, 0)),
        ],
        out_shape=[
            jax.ShapeDtypeStruct((n, 32), jnp.float32),
            jax.ShapeDtypeStruct((n, 128), jnp.float32),
        ],
    )(*([g] * kp), Wq, bq.reshape(1, -1), Wk, bk.reshape(1, -1),
      Wv, bv.reshape(1, -1))


def _topk_body(q_ref, xvt_ref, o_ref, *, k, npad, rounds):
    qb = q_ref[...]
    xvt = xvt_ref[...]
    qn = qb.shape[0]
    sq = jnp.sum(xvt * xvt, axis=0, keepdims=True)
    qsq = jnp.sum(qb * qb, axis=1, keepdims=True)
    d2 = qsq - 2.0 * jnp.dot(qb, xvt, preferred_element_type=jnp.float32) + sq
    nc = npad // 128
    d2r = d2.reshape(qn, nc, 128)
    ig = (lax.broadcasted_iota(jnp.int32, (1, nc, 128), 1) * 128
          + lax.broadcasted_iota(jnp.int32, (1, nc, 128), 2)).astype(jnp.float32)
    vals, idxs = [], []
    for _ in range(rounds):
        mc = jnp.min(d2r, axis=1, keepdims=True)
        cand = jnp.where(d2r == mc, ig, _BIG)
        ic = jnp.min(cand, axis=1, keepdims=True)
        vals.append(mc.reshape(qn, 128))
        idxs.append(ic.reshape(qn, 128))
        d2r = jnp.where(ig == ic, _BIG, d2r)
    v = jnp.concatenate(vals, axis=1)
    iv = jnp.concatenate(idxs, axis=1)
    cols = []
    for _ in range(k):
        m = jnp.min(v, axis=1, keepdims=True)
        cand = jnp.where(v == m, iv, _BIG)
        idxf = jnp.min(cand, axis=1, keepdims=True)
        cols.append(idxf)
        v = jnp.where(iv == idxf, _BIG, v)
    cols.append(jnp.zeros((qn, 32 - k), jnp.float32))
    o_ref[...] = jnp.concatenate(cols, axis=1).astype(jnp.int32)


def _knn_topk(xvp, xvt, qb, npad):
    grid = npad // qb
    return pl.pallas_call(
        functools.partial(_topk_body, k=KNN_K, npad=npad, rounds=5),
        grid=(grid,),
        in_specs=[
            pl.BlockSpec((qb, 3), lambda i: (i, 0)),
            pl.BlockSpec((3, npad), lambda i: (0, 0)),
        ],
        out_specs=pl.BlockSpec((qb, 32), lambda i: (i, 0)),
        out_shape=jax.ShapeDtypeStruct((npad, 32), jnp.int32),
    )(xvp, xvt)


def _mha_body(*refs):
    kv = [r[...] for r in refs[:KNN_K]]
    hq_ref, wo_ref, bo_ref, wout_ref, o_ref = refs[KNN_K:]
    hq = hq_ref[...]
    q = hq[:, 16:32]
    s = [q * kvj[:, 0:16] for kvj in kv]
    m = s[0]
    for sj in s[1:]:
        m = jnp.maximum(m, sj)
    e = [jnp.exp(sj - m) for sj in s]
    z = e[0]
    for ej in e[1:]:
        z = z + ej
    acc = e[0] * kv[0][:, 16:32]
    for ej, kvj in zip(e[1:], kv[1:]):
        acc = acc + ej * kvj[:, 16:32]
    o = acc / z
    res = hq[:, 0:16] + jnp.dot(o, wo_ref[...],
                                preferred_element_type=jnp.float32) + bo_ref[...]
    o_ref[...] = jnp.dot(res, wout_ref[...], preferred_element_type=jnp.float32)


def _mha_out(kvg, hq, Wo, bo, Wout, pb):
    n = hq.shape[0]
    nb = n // pb
    kvspecs = [pl.BlockSpec((pb, 128), functools.partial(
        lambda i, j: (j * nb + i, 0), j=j)) for j in range(KNN_K)]
    return pl.pallas_call(
        _mha_body,
        grid=(nb,),
        in_specs=kvspecs + [
            pl.BlockSpec((pb, 32), lambda i: (i, 0)),
            pl.BlockSpec((16, 16), lambda i: (0, 0)),
            pl.BlockSpec((1, 16), lambda i: (0, 0)),
            pl.BlockSpec((16, 1), lambda i: (0, 0)),
        ],
        out_specs=pl.BlockSpec((pb, 1), lambda i: (i, 0)),
        out_shape=jax.ShapeDtypeStruct((n, 1), jnp.float32),
    )(*([kvg] * KNN_K), hq, Wo, bo.reshape(1, -1), Wout)


def kernel(x, x_v, unpooling_idx, W_embed, W1, b1, Vw, Vb,
           Wq, bq, Wk, bk, Wv, bv, Wo, bo, Wout):
    n_in = x.shape[1]
    n_out = x_v.shape[1]
    kp = unpooling_idx.shape[2]

    x2 = x.reshape(n_in, 3)
    xv2 = x_v.reshape(n_out, 3)

    h0s = _embed(x2, W_embed, W1, b1, Vw, Vb)

    uidx = jnp.transpose(unpooling_idx.reshape(n_out, kp)).reshape(
        n_out * kp).astype(jnp.int32)
    g = _sc_gather_rows(h0s, uidx)

    pb = 400 if n_out % 400 == 0 else n_out
    hq, kv = _pool_proj(g, kp, n_out, Wq, bq, Wk, bk, Wv, bv, pb)

    qb = 256
    npad = -(-n_out // 512) * 512
    pad = jnp.full((npad - n_out, 3), 1e4, jnp.float32)
    xvp = jnp.concatenate([xv2, pad], axis=0)
    xvt = xvp.T
    knn = _knn_topk(xvp, xvt, qb, npad)
    kidx = jnp.transpose(knn[:n_out, :KNN_K]).reshape(
        KNN_K * n_out)

    kvg = _sc_gather_rows(kv, kidx)

    out = _mha_out(kvg, hq, Wo, bo, Wout, pb)
    return out.reshape(1, n_out, 1)


# speedup vs baseline: 1.3384x; 1.1038x over previous
import functools

import jax
import jax.numpy as jnp
from jax import lax
from jax.experimental import pallas as pl
from jax.experimental.pallas import tpu as pltpu
from jax.experimental.pallas import tpu_sc as plsc

KNN_K = 27
_BIG = 1e30


def _embed_body(x_ref, we_ref, w1_ref, b1_ref, vw_ref, vb_ref, o_ref):
    h0 = jnp.dot(x_ref[...], we_ref[...], preferred_element_type=jnp.float32)
    t = jnp.tanh(jnp.dot(h0, w1_ref[...], preferred_element_type=jnp.float32)
                 + b1_ref[...])
    s0 = jnp.dot(t, vw_ref[...], preferred_element_type=jnp.float32) + vb_ref[...]
    n = h0.shape[0]
    o_ref[...] = jnp.concatenate(
        [h0, s0, jnp.zeros((n, 111), jnp.float32)], axis=1)


def _embed(x, W_embed, W1, b1, Vw, Vb):
    n = x.shape[0]
    return pl.pallas_call(
        _embed_body,
        out_shape=jax.ShapeDtypeStruct((n, 128), jnp.float32),
    )(x, W_embed, W1, b1.reshape(1, -1), Vw, Vb.reshape(1, 1))


def _sc_gather_rows(table, idx_flat):
    nidx = idx_flat.shape[0]
    n_chunk = -(-nidx // 4096)
    bpad = n_chunk * 4096
    idx2 = jnp.concatenate(
        [idx_flat, jnp.zeros((bpad - nidx,), jnp.int32)]).reshape(32 * n_chunk, 128)

    mesh = plsc.VectorSubcoreMesh(core_axis_name="c", subcore_axis_name="s")

    @functools.partial(
        pl.kernel, mesh=mesh,
        out_type=jax.ShapeDtypeStruct((bpad, 128), jnp.float32),
        scratch_types=[
            pltpu.VMEM((2, 128), jnp.int32),
            pltpu.VMEM((2, 128, 128), jnp.float32),
            pltpu.SemaphoreType.DMA,
            pltpu.SemaphoreType.DMA,
        ],
    )
    def gk(table_hbm, idx_hbm, out_hbm, idx_v, rows_v, s_g, s_o):
        wid = lax.axis_index("s") * 2 + lax.axis_index("c")
        base = wid * n_chunk

        pltpu.sync_copy(idx_hbm.at[base], idx_v.at[0])
        pltpu.async_copy(table_hbm.at[idx_v.at[0]], rows_v.at[0], s_g)

        def body(c, carry):
            cur = lax.rem(c, 2)
            nxt = 1 - cur

            @pl.when(c + 1 < n_chunk)
            def _():
                @pl.when(c >= 1)
                def _():
                    pltpu.make_async_copy(
                        rows_v.at[nxt],
                        out_hbm.at[pl.ds((base + c - 1) * 128, 128)],
                        s_o).wait()
                pltpu.sync_copy(idx_hbm.at[base + c + 1], idx_v.at[nxt])
                pltpu.async_copy(
                    table_hbm.at[idx_v.at[nxt]], rows_v.at[nxt], s_g)

            pltpu.make_async_copy(
                table_hbm.at[idx_v.at[cur]], rows_v.at[cur], s_g).wait()
            pltpu.async_copy(
                rows_v.at[cur], out_hbm.at[pl.ds((base + c) * 128, 128)], s_o)
            return carry

        lax.fori_loop(0, n_chunk, body, 0)
        last = lax.rem(n_chunk - 1, 2)
        pltpu.make_async_copy(
            rows_v.at[1 - last],
            out_hbm.at[pl.ds((base + n_chunk - 2) * 128, 128)], s_o).wait()
        pltpu.make_async_copy(
            rows_v.at[last],
            out_hbm.at[pl.ds((base + n_chunk - 1) * 128, 128)], s_o).wait()

    return gk(table, idx2)


def _pool_body(*refs):
    g = refs[:-8]
    wq_ref, bq_ref, wk_ref, bk_ref, wv_ref, bv_ref, hq_ref, kv_ref = refs[-8:]
    gv = [r[...] for r in g]
    s = [gj[:, 16:17] for gj in gv]
    m = s[0]
    for sj in s[1:]:
        m = jnp.maximum(m, sj)
    e = [jnp.exp(sj - m) for sj in s]
    z = e[0]
    for ej in e[1:]:
        z = z + ej
    acc = e[0] * gv[0][:, 0:16]
    for ej, gj in zip(e[1:], gv[1:]):
        acc = acc + ej * gj[:, 0:16]
    h1 = acc / z
    q = jnp.dot(h1, wq_ref[...], preferred_element_type=jnp.float32) + bq_ref[...]
    k = jnp.dot(h1, wk_ref[...], preferred_element_type=jnp.float32) + bk_ref[...]
    v = jnp.dot(h1, wv_ref[...], preferred_element_type=jnp.float32) + bv_ref[...]
    pb = h1.shape[0]
    hq_ref[...] = jnp.concatenate([h1, q], axis=1)
    kv_ref[...] = jnp.concatenate(
        [k, v, jnp.zeros((pb, 96), jnp.float32)], axis=1)


def _pool_proj(g, kp, n, Wq, bq, Wk, bk, Wv, bv, pb):
    nb = n // pb
    wspec = pl.BlockSpec((16, 16), lambda i: (0, 0))
    bspec = pl.BlockSpec((1, 16), lambda i: (0, 0))
    gspecs = [pl.BlockSpec((pb, 128), functools.partial(
        lambda i, j: (j * nb + i, 0), j=j)) for j in range(kp)]
    return pl.pallas_call(
        _pool_body,
        grid=(nb,),
        in_specs=gspecs + [wspec, bspec, wspec, bspec, wspec, bspec],
        out_specs=[
            pl.BlockSpec((pb, 32), lambda i: (i, 0)),
            pl.BlockSpec((pb, 128), lambda i: (i, 0)),
        ],
        out_shape=[
            jax.ShapeDtypeStruct((n, 32), jnp.float32),
            jax.ShapeDtypeStruct((n, 128), jnp.float32),
        ],
    )(*([g] * kp), Wq, bq.reshape(1, -1), Wk, bk.reshape(1, -1),
      Wv, bv.reshape(1, -1))


def _topk_body(q_ref, xvt_ref, o_ref, *, k, npad, rounds):
    qb = q_ref[...]
    xvt = xvt_ref[...]
    qn = qb.shape[0]
    sq = jnp.sum(xvt * xvt, axis=0, keepdims=True)
    qsq = jnp.sum(qb * qb, axis=1, keepdims=True)
    d2 = qsq - 2.0 * jnp.dot(qb, xvt, preferred_element_type=jnp.float32) + sq
    nc = npad // 128
    d2r = d2.reshape(qn, nc, 128)
    ig = (lax.broadcasted_iota(jnp.int32, (1, nc, 128), 1) * 128
          + lax.broadcasted_iota(jnp.int32, (1, nc, 128), 2)).astype(jnp.float32)
    vals, idxs = [], []
    for _ in range(rounds):
        mc = jnp.min(d2r, axis=1, keepdims=True)
        cand = jnp.where(d2r == mc, ig, _BIG)
        ic = jnp.min(cand, axis=1, keepdims=True)
        vals.append(mc.reshape(qn, 128))
        idxs.append(ic.reshape(qn, 128))
        d2r = jnp.where(ig == ic, _BIG, d2r)
    v = jnp.concatenate(vals, axis=1)
    iv = jnp.concatenate(idxs, axis=1)
    cols = []
    for _ in range(k):
        m = jnp.min(v, axis=1, keepdims=True)
        cand = jnp.where(v == m, iv, _BIG)
        idxf = jnp.min(cand, axis=1, keepdims=True)
        cols.append(idxf)
        v = jnp.where(iv == idxf, _BIG, v)
    cols.append(jnp.zeros((qn, 32 - k), jnp.float32))
    o_ref[...] = jnp.concatenate(cols, axis=1).astype(jnp.int32)


def _knn_topk(xvp, xvt, qb, npad):
    grid = npad // qb
    return pl.pallas_call(
        functools.partial(_topk_body, k=KNN_K, npad=npad, rounds=4),
        grid=(grid,),
        in_specs=[
            pl.BlockSpec((qb, 3), lambda i: (i, 0)),
            pl.BlockSpec((3, npad), lambda i: (0, 0)),
        ],
        out_specs=pl.BlockSpec((qb, 32), lambda i: (i, 0)),
        out_shape=jax.ShapeDtypeStruct((npad, 32), jnp.int32),
    )(xvp, xvt)


def _mha_body(*refs):
    kv = [r[...] for r in refs[:KNN_K]]
    hq_ref, wo_ref, bo_ref, wout_ref, o_ref = refs[KNN_K:]
    hq = hq_ref[...]
    q = hq[:, 16:32]
    s = [q * kvj[:, 0:16] for kvj in kv]
    m = s[0]
    for sj in s[1:]:
        m = jnp.maximum(m, sj)
    e = [jnp.exp(sj - m) for sj in s]
    z = e[0]
    for ej in e[1:]:
        z = z + ej
    acc = e[0] * kv[0][:, 16:32]
    for ej, kvj in zip(e[1:], kv[1:]):
        acc = acc + ej * kvj[:, 16:32]
    o = acc / z
    res = hq[:, 0:16] + jnp.dot(o, wo_ref[...],
                                preferred_element_type=jnp.float32) + bo_ref[...]
    o_ref[...] = jnp.dot(res, wout_ref[...], preferred_element_type=jnp.float32)


def _mha_out(kvg, hq, Wo, bo, Wout, pb):
    n = hq.shape[0]
    nb = n // pb
    kvspecs = [pl.BlockSpec((pb, 128), functools.partial(
        lambda i, j: (j * nb + i, 0), j=j)) for j in range(KNN_K)]
    return pl.pallas_call(
        _mha_body,
        grid=(nb,),
        in_specs=kvspecs + [
            pl.BlockSpec((pb, 32), lambda i: (i, 0)),
            pl.BlockSpec((16, 16), lambda i: (0, 0)),
            pl.BlockSpec((1, 16), lambda i: (0, 0)),
            pl.BlockSpec((16, 1), lambda i: (0, 0)),
        ],
        out_specs=pl.BlockSpec((pb, 1), lambda i: (i, 0)),
        out_shape=jax.ShapeDtypeStruct((n, 1), jnp.float32),
    )(*([kvg] * KNN_K), hq, Wo, bo.reshape(1, -1), Wout)


def kernel(x, x_v, unpooling_idx, W_embed, W1, b1, Vw, Vb,
           Wq, bq, Wk, bk, Wv, bv, Wo, bo, Wout):
    n_in = x.shape[1]
    n_out = x_v.shape[1]
    kp = unpooling_idx.shape[2]

    x2 = x.reshape(n_in, 3)
    xv2 = x_v.reshape(n_out, 3)

    h0s = _embed(x2, W_embed, W1, b1, Vw, Vb)

    uidx = jnp.transpose(unpooling_idx.reshape(n_out, kp)).reshape(
        n_out * kp).astype(jnp.int32)
    g = _sc_gather_rows(h0s, uidx)

    pb = 400 if n_out % 400 == 0 else n_out
    hq, kv = _pool_proj(g, kp, n_out, Wq, bq, Wk, bk, Wv, bv, pb)

    qb = 256
    npad = -(-n_out // 512) * 512
    pad = jnp.full((npad - n_out, 3), 1e4, jnp.float32)
    xvp = jnp.concatenate([xv2, pad], axis=0)
    xvt = xvp.T
    knn = _knn_topk(xvp, xvt, qb, npad)
    kidx = jnp.transpose(knn[:n_out, :KNN_K]).reshape(
        KNN_K * n_out)

    kvg = _sc_gather_rows(kv, kidx)

    out = _mha_out(kvg, hq, Wo, bo, Wout, pb)
    return out.reshape(1, n_out, 1)


# speedup vs baseline: 1.3551x; 1.0125x over previous
import functools

import jax
import jax.numpy as jnp
from jax import lax
from jax.experimental import pallas as pl
from jax.experimental.pallas import tpu as pltpu
from jax.experimental.pallas import tpu_sc as plsc

KNN_K = 27
_BIG = 1e30


def _embed_body(x_ref, we_ref, w1_ref, b1_ref, vw_ref, vb_ref, o_ref):
    h0 = jnp.dot(x_ref[...], we_ref[...], preferred_element_type=jnp.float32)
    t = jnp.tanh(jnp.dot(h0, w1_ref[...], preferred_element_type=jnp.float32)
                 + b1_ref[...])
    s0 = jnp.dot(t, vw_ref[...], preferred_element_type=jnp.float32) + vb_ref[...]
    n = h0.shape[0]
    o_ref[...] = jnp.concatenate(
        [h0, s0, jnp.zeros((n, 111), jnp.float32)], axis=1)


def _embed(x, W_embed, W1, b1, Vw, Vb):
    n = x.shape[0]
    return pl.pallas_call(
        _embed_body,
        out_shape=jax.ShapeDtypeStruct((n, 128), jnp.float32),
    )(x, W_embed, W1, b1.reshape(1, -1), Vw, Vb.reshape(1, 1))


def _sc_gather_rows(table, idx_flat):
    nidx = idx_flat.shape[0]
    n_chunk = -(-nidx // 4096)
    bpad = n_chunk * 4096
    idx2 = jnp.concatenate(
        [idx_flat, jnp.zeros((bpad - nidx,), jnp.int32)]).reshape(32, n_chunk, 128)

    mesh = plsc.VectorSubcoreMesh(core_axis_name="c", subcore_axis_name="s")

    @functools.partial(
        pl.kernel, mesh=mesh,
        out_type=jax.ShapeDtypeStruct((bpad, 128), jnp.float32),
        scratch_types=[
            pltpu.VMEM((n_chunk, 128), jnp.int32),
            pltpu.VMEM((4, 128, 128), jnp.float32),
            pltpu.SemaphoreType.DMA,
            pltpu.SemaphoreType.DMA,
        ],
    )
    def gk(table_hbm, idx_hbm, out_hbm, idx_all, rows_v, s_g, s_o):
        wid = lax.axis_index("s") * 2 + lax.axis_index("c")
        base = wid * n_chunk

        pltpu.sync_copy(idx_hbm.at[wid], idx_all)
        for b in range(3):
            pltpu.async_copy(table_hbm.at[idx_all.at[b]], rows_v.at[b], s_g)

        def body(c, carry):
            cur = lax.rem(c, 4)

            @pl.when(c + 3 < n_chunk)
            def _():
                @pl.when(c >= 1)
                def _():
                    pltpu.make_async_copy(
                        rows_v.at[lax.rem(c + 3, 4)],
                        out_hbm.at[pl.ds((base + c - 1) * 128, 128)],
                        s_o).wait()
                pltpu.async_copy(
                    table_hbm.at[idx_all.at[c + 3]],
                    rows_v.at[lax.rem(c + 3, 4)], s_g)

            pltpu.make_async_copy(
                table_hbm.at[idx_all.at[c]], rows_v.at[cur], s_g).wait()
            pltpu.async_copy(
                rows_v.at[cur], out_hbm.at[pl.ds((base + c) * 128, 128)], s_o)
            return carry

        lax.fori_loop(0, n_chunk, body, 0)
        for t in range(4):
            c = n_chunk - 4 + t
            pltpu.make_async_copy(
                rows_v.at[lax.rem(c, 4)],
                out_hbm.at[pl.ds((base + c) * 128, 128)], s_o).wait()

    return gk(table, idx2)


def _pool_body(*refs):
    g = refs[:-8]
    wq_ref, bq_ref, wk_ref, bk_ref, wv_ref, bv_ref, hq_ref, kv_ref = refs[-8:]
    gv = [r[...] for r in g]
    s = [gj[:, 16:17] for gj in gv]
    m = s[0]
    for sj in s[1:]:
        m = jnp.maximum(m, sj)
    e = [jnp.exp(sj - m) for sj in s]
    z = e[0]
    for ej in e[1:]:
        z = z + ej
    acc = e[0] * gv[0][:, 0:16]
    for ej, gj in zip(e[1:], gv[1:]):
        acc = acc + ej * gj[:, 0:16]
    h1 = acc / z
    q = jnp.dot(h1, wq_ref[...], preferred_element_type=jnp.float32) + bq_ref[...]
    k = jnp.dot(h1, wk_ref[...], preferred_element_type=jnp.float32) + bk_ref[...]
    v = jnp.dot(h1, wv_ref[...], preferred_element_type=jnp.float32) + bv_ref[...]
    pb = h1.shape[0]
    hq_ref[...] = jnp.concatenate([h1, q], axis=1)
    kv_ref[...] = jnp.concatenate(
        [k, v, jnp.zeros((pb, 96), jnp.float32)], axis=1)


def _pool_proj(g, kp, n, Wq, bq, Wk, bk, Wv, bv, pb):
    nb = n // pb
    wspec = pl.BlockSpec((16, 16), lambda i: (0, 0))
    bspec = pl.BlockSpec((1, 16), lambda i: (0, 0))
    gspecs = [pl.BlockSpec((pb, 128), functools.partial(
        lambda i, j: (j * nb + i, 0), j=j)) for j in range(kp)]
    return pl.pallas_call(
        _pool_body,
        grid=(nb,),
        in_specs=gspecs + [wspec, bspec, wspec, bspec, wspec, bspec],
        out_specs=[
            pl.BlockSpec((pb, 32), lambda i: (i, 0)),
            pl.BlockSpec((pb, 128), lambda i: (i, 0)),
        ],
        out_shape=[
            jax.ShapeDtypeStruct((n, 32), jnp.float32),
            jax.ShapeDtypeStruct((n, 128), jnp.float32),
        ],
    )(*([g] * kp), Wq, bq.reshape(1, -1), Wk, bk.reshape(1, -1),
      Wv, bv.reshape(1, -1))


def _topk_body(q_ref, xvt_ref, o_ref, *, k, npad, rounds):
    qb = q_ref[...]
    xvt = xvt_ref[...]
    qn = qb.shape[0]
    sq = jnp.sum(xvt * xvt, axis=0, keepdims=True)
    qsq = jnp.sum(qb * qb, axis=1, keepdims=True)
    d2 = qsq - 2.0 * jnp.dot(qb, xvt, preferred_element_type=jnp.float32) + sq
    nc = npad // 128
    d2r = d2.reshape(qn, nc, 128)
    ig = (lax.broadcasted_iota(jnp.int32, (1, nc, 128), 1) * 128
          + lax.broadcasted_iota(jnp.int32, (1, nc, 128), 2)).astype(jnp.float32)
    vals, idxs = [], []
    for _ in range(rounds):
        mc = jnp.min(d2r, axis=1, keepdims=True)
        cand = jnp.where(d2r == mc, ig, _BIG)
        ic = jnp.min(cand, axis=1, keepdims=True)
        vals.append(mc.reshape(qn, 128))
        idxs.append(ic.reshape(qn, 128))
        d2r = jnp.where(ig == ic, _BIG, d2r)
    v = jnp.concatenate(vals, axis=1)
    iv = jnp.concatenate(idxs, axis=1)
    cols = []
    for _ in range(k):
        m = jnp.min(v, axis=1, keepdims=True)
        cand = jnp.where(v == m, iv, _BIG)
        idxf = jnp.min(cand, axis=1, keepdims=True)
        cols.append(idxf)
        v = jnp.where(iv == idxf, _BIG, v)
    cols.append(jnp.zeros((qn, 32 - k), jnp.float32))
    o_ref[...] = jnp.concatenate(cols, axis=1).astype(jnp.int32)


def _knn_topk(xvp, xvt, qb, npad):
    grid = npad // qb
    return pl.pallas_call(
        functools.partial(_topk_body, k=KNN_K, npad=npad, rounds=4),
        grid=(grid,),
        in_specs=[
            pl.BlockSpec((qb, 3), lambda i: (i, 0)),
            pl.BlockSpec((3, npad), lambda i: (0, 0)),
        ],
        out_specs=pl.BlockSpec((qb, 32), lambda i: (i, 0)),
        out_shape=jax.ShapeDtypeStruct((npad, 32), jnp.int32),
    )(xvp, xvt)


def _mha_body(*refs):
    kv = [r[...] for r in refs[:KNN_K]]
    hq_ref, wo_ref, bo_ref, wout_ref, o_ref = refs[KNN_K:]
    hq = hq_ref[...]
    q = hq[:, 16:32]
    s = [q * kvj[:, 0:16] for kvj in kv]
    m = s[0]
    for sj in s[1:]:
        m = jnp.maximum(m, sj)
    e = [jnp.exp(sj - m) for sj in s]
    z = e[0]
    for ej in e[1:]:
        z = z + ej
    acc = e[0] * kv[0][:, 16:32]
    for ej, kvj in zip(e[1:], kv[1:]):
        acc = acc + ej * kvj[:, 16:32]
    o = acc / z
    res = hq[:, 0:16] + jnp.dot(o, wo_ref[...],
                                preferred_element_type=jnp.float32) + bo_ref[...]
    o_ref[...] = jnp.dot(res, wout_ref[...], preferred_element_type=jnp.float32)


def _mha_out(kvg, hq, Wo, bo, Wout, pb):
    n = hq.shape[0]
    nb = n // pb
    kvspecs = [pl.BlockSpec((pb, 128), functools.partial(
        lambda i, j: (j * nb + i, 0), j=j)) for j in range(KNN_K)]
    return pl.pallas_call(
        _mha_body,
        grid=(nb,),
        in_specs=kvspecs + [
            pl.BlockSpec((pb, 32), lambda i: (i, 0)),
            pl.BlockSpec((16, 16), lambda i: (0, 0)),
            pl.BlockSpec((1, 16), lambda i: (0, 0)),
            pl.BlockSpec((16, 1), lambda i: (0, 0)),
        ],
        out_specs=pl.BlockSpec((pb, 1), lambda i: (i, 0)),
        out_shape=jax.ShapeDtypeStruct((n, 1), jnp.float32),
    )(*([kvg] * KNN_K), hq, Wo, bo.reshape(1, -1), Wout)


def kernel(x, x_v, unpooling_idx, W_embed, W1, b1, Vw, Vb,
           Wq, bq, Wk, bk, Wv, bv, Wo, bo, Wout):
    n_in = x.shape[1]
    n_out = x_v.shape[1]
    kp = unpooling_idx.shape[2]

    x2 = x.reshape(n_in, 3)
    xv2 = x_v.reshape(n_out, 3)

    h0s = _embed(x2, W_embed, W1, b1, Vw, Vb)

    uidx = jnp.transpose(unpooling_idx.reshape(n_out, kp)).reshape(
        n_out * kp).astype(jnp.int32)
    g = _sc_gather_rows(h0s, uidx)

    pb = 400 if n_out % 400 == 0 else n_out
    hq, kv = _pool_proj(g, kp, n_out, Wq, bq, Wk, bk, Wv, bv, pb)

    qb = 256
    npad = -(-n_out // 512) * 512
    pad = jnp.full((npad - n_out, 3), 1e4, jnp.float32)
    xvp = jnp.concatenate([xv2, pad], axis=0)
    xvt = xvp.T
    knn = _knn_topk(xvp, xvt, qb, npad)
    kidx = jnp.transpose(knn[:n_out, :KNN_K]).reshape(
        KNN_K * n_out)

    kvg = _sc_gather_rows(kv, kidx)

    out = _mha_out(kvg, hq, Wo, bo, Wout, pb)
    return out.reshape(1, n_out, 1)
